# Initial kernel scaffold; baseline (speedup 1.0000x reference)
import jax
import jax.numpy as jnp
from jax.experimental import pallas as pl


def kernel(x, edge_index, conv_w, conv_b, W0, asrc0, adst0, b0, W1, asrc1, adst1, b1, ln_w, ln_b):
    raise NotImplementedError("write your pallas kernel here")


# speedup vs baseline: 1.8033x; 1.8033x over previous
import functools

import jax
import jax.numpy as jnp
from jax import lax
from jax.experimental import pallas as pl

_INTERPRET = False

N_NODES = 10000
B, T, H = 5, 2000, 128
E = 320000


def _stage_a_body(x_ref, w0t_ref, w1t_ref, w2t_ref, cb_ref, W0_ref, as0_ref,
                  ad0_ref, h0_ref, asv_ref, adv_ref):
    x2 = x_ref[0]
    c0 = jnp.dot(x2, w0t_ref[...], preferred_element_type=jnp.float32)
    c1 = jnp.dot(x2, w1t_ref[...], preferred_element_type=jnp.float32)
    c2 = jnp.dot(x2, w2t_ref[...], preferred_element_type=jnp.float32)
    z1 = jnp.zeros((1, H), jnp.float32)
    conv = (c1 + jnp.concatenate([z1, c0[:-1]], axis=0)
            + jnp.concatenate([c2[1:], z1], axis=0) + cb_ref[...])
    y = jax.nn.relu(conv + x2)
    h0 = jnp.dot(y, W0_ref[...], preferred_element_type=jnp.float32)
    h0_ref[0] = h0
    asv_ref[0] = jnp.dot(h0, as0_ref[...], preferred_element_type=jnp.float32).T
    adv_ref[0] = jnp.dot(h0, ad0_ref[...], preferred_element_type=jnp.float32).T


def _stage_a(x2, w0t, w1t, w2t, cb, W0, as0, ad0):
    return pl.pallas_call(
        _stage_a_body,
        grid=(B,),
        in_specs=[
            pl.BlockSpec((1, T, H), lambda b: (b, 0, 0)),
            pl.BlockSpec((H, H), lambda b: (0, 0)),
            pl.BlockSpec((H, H), lambda b: (0, 0)),
            pl.BlockSpec((H, H), lambda b: (0, 0)),
            pl.BlockSpec((1, H), lambda b: (0, 0)),
            pl.BlockSpec((H, H), lambda b: (0, 0)),
            pl.BlockSpec((H, 1), lambda b: (0, 0)),
            pl.BlockSpec((H, 1), lambda b: (0, 0)),
        ],
        out_specs=[
            pl.BlockSpec((1, T, H), lambda b: (b, 0, 0)),
            pl.BlockSpec((1, 1, T), lambda b: (b, 0, 0)),
            pl.BlockSpec((1, 1, T), lambda b: (b, 0, 0)),
        ],
        out_shape=[
            jax.ShapeDtypeStruct((B, T, H), jnp.float32),
            jax.ShapeDtypeStruct((B, 1, T), jnp.float32),
            jax.ShapeDtypeStruct((B, 1, T), jnp.float32),
        ],
        interpret=_INTERPRET,
    )(x2, w0t, w1t, w2t, cb, W0, as0, ad0)


def _stage_c_body(acc_ref, den_ref, b0_ref, W1_ref, as1_ref, ad1_ref,
                  z_ref, asv_ref, adv_ref):
    acc = acc_ref[0] + acc_ref[1]
    d = den_ref[0, 0] + den_ref[1, 0]
    dsafe = jnp.where(d > 0, d, 1.0)
    z = jax.nn.relu(acc / dsafe[:, None] + b0_ref[...])
    z_ref[...] = z
    w_as = jnp.dot(W1_ref[...], as1_ref[...], preferred_element_type=jnp.float32)
    w_ad = jnp.dot(W1_ref[...], ad1_ref[...], preferred_element_type=jnp.float32)
    asv_ref[...] = jnp.dot(z, w_as, preferred_element_type=jnp.float32).T
    adv_ref[...] = jnp.dot(z, w_ad, preferred_element_type=jnp.float32).T


def _stage_c(accum2, denom2, b0, W1, as1, ad1, n_pad, rows):
    grid = n_pad // rows
    return pl.pallas_call(
        _stage_c_body,
        grid=(grid,),
        in_specs=[
            pl.BlockSpec((2, rows, H), lambda r: (0, r, 0)),
            pl.BlockSpec((2, 1, rows), lambda r: (0, 0, r)),
            pl.BlockSpec((1, H), lambda r: (0, 0)),
            pl.BlockSpec((H, H), lambda r: (0, 0)),
            pl.BlockSpec((H, 1), lambda r: (0, 0)),
            pl.BlockSpec((H, 1), lambda r: (0, 0)),
        ],
        out_specs=[
            pl.BlockSpec((rows, H), lambda r: (r, 0)),
            pl.BlockSpec((1, rows), lambda r: (0, r)),
            pl.BlockSpec((1, rows), lambda r: (0, r)),
        ],
        out_shape=[
            jax.ShapeDtypeStruct((n_pad, H), jnp.float32),
            jax.ShapeDtypeStruct((1, n_pad), jnp.float32),
            jax.ShapeDtypeStruct((1, n_pad), jnp.float32),
        ],
        interpret=_INTERPRET,
    )(accum2, denom2.reshape(2, 1, n_pad), b0, W1, as1, ad1)


def _stage_e_body(cf_ref, z_ref, W1_ref, b1_ref, w1t_ref, w2t_ref, cb_ref,
                  lnw_ref, lnb_ref, out_ref):
    coeff = cf_ref[0] + cf_ref[1]
    denom1 = coeff.sum(-1, keepdims=True)
    pre = jnp.dot(coeff, z_ref[...], preferred_element_type=jnp.float32)
    d1safe = jnp.where(denom1 > 0, denom1, 1.0)
    out1 = jnp.dot(pre / d1safe, W1_ref[...],
                   preferred_element_type=jnp.float32) + b1_ref[...]
    g = jax.nn.relu(out1)
    g0 = g[0:5]
    g1 = g[5:10]
    f = (jnp.dot(g0, w1t_ref[...], preferred_element_type=jnp.float32)
         + jnp.dot(g1, w2t_ref[...], preferred_element_type=jnp.float32)
         + cb_ref[...])
    mean = f.mean(-1, keepdims=True)
    var = ((f - mean) ** 2).mean(-1, keepdims=True)
    f = (f - mean) * jax.lax.rsqrt(var + 1e-5) * lnw_ref[...] + lnb_ref[...]
    out_ref[...] = jax.nn.relu(f)


def _stage_e(coeff2, z, W1, b1, w1t, w2t, cb, lnw, lnb, n_pad):
    return pl.pallas_call(
        _stage_e_body,
        out_shape=jax.ShapeDtypeStruct((B, H), jnp.float32),
        interpret=_INTERPRET,
    )(coeff2, z, W1, b1, w1t, w2t, cb, lnw, lnb)


def _edge_phase_gat0(a_s0, a_d0, h0, src, dst, n_pad):
    e0 = a_s0[src] + a_d0[dst]
    e0 = jnp.where(e0 >= 0, e0, 0.2 * e0)
    ex0 = jnp.exp(e0)
    denom = jax.ops.segment_sum(ex0, dst, num_segments=n_pad)
    accum = jax.ops.segment_sum(ex0[:, None] * h0[src], dst, num_segments=n_pad)
    return accum[None], denom[None]


def _edge_phase_gat1(a_s1, a_d1, src, dst, n_pad):
    tmod = dst % T
    mask = tmod < 2
    tgt = tmod * B + dst // T
    e1 = a_s1[src] + a_d1[dst]
    e1 = jnp.where(e1 >= 0, e1, 0.2 * e1)
    ex1 = jnp.where(mask, jnp.exp(e1), 0.0)
    flat = jnp.where(mask, tgt * n_pad + src, 10 * n_pad + src)
    coeff = jax.ops.segment_sum(ex1, flat, num_segments=16 * n_pad)
    return coeff.reshape(1, 16, n_pad)


def kernel(x, edge_index, conv_w, conv_b, W0, asrc0, adst0, b0, W1, asrc1,
           adst1, b1, ln_w, ln_b):
    n_pad = 10240
    x2 = x[:, 0]
    w0t = conv_w[:, :, 0].T
    w1t = conv_w[:, :, 1].T
    w2t = conv_w[:, :, 2].T
    cb = conv_b[None, :]
    src, dst = edge_index[0], edge_index[1]

    h0, a_s0, a_d0 = _stage_a(x2, w0t, w1t, w2t, cb, W0,
                              asrc0[:, None], adst0[:, None])
    h0 = h0.reshape(N_NODES, H)
    a_s0 = a_s0.reshape(N_NODES)
    a_d0 = a_d0.reshape(N_NODES)

    acc1, den1 = _edge_phase_gat0(a_s0, a_d0, h0, src, dst, n_pad)
    accum2 = jnp.concatenate([acc1, jnp.zeros_like(acc1)], axis=0)
    denom2 = jnp.concatenate([den1, jnp.zeros_like(den1)], axis=0)

    z, a_s1, a_d1 = _stage_c(accum2, denom2, b0[None, :], W1,
                             asrc1[:, None], adst1[:, None], n_pad, 1280)
    a_s1 = a_s1.reshape(n_pad)
    a_d1 = a_d1.reshape(n_pad)

    cf1 = _edge_phase_gat1(a_s1, a_d1, src, dst, n_pad)
    coeff2 = jnp.concatenate([cf1, jnp.zeros_like(cf1)], axis=0)

    return _stage_e(coeff2, z, W1, b1, w1t, w2t, cb, ln_w[None, :],
                    ln_b[None, :], n_pad)
